# baseline (speedup 1.0000x reference)
import jax
import jax.numpy as jnp
from jax.experimental import pallas as pl
from jax.experimental.pallas import tpu as pltpu

_D = 128
_K = 32
_H = 64
_W = 64


def _enc_kernel(x_ref, cw_ref, scale_ref, out_ref):
    Xb = x_ref[0].reshape(_D, _H * _W)
    C = cw_ref[...]
    s = scale_ref[...]
    c2 = jnp.sum(C * C, axis=1, keepdims=True)
    x2 = jnp.sum(Xb * Xb, axis=0, keepdims=True)
    xc = jax.lax.dot_general(C, Xb, (((1,), (0,)), ((), ())),
                             preferred_element_type=jnp.float32)
    SL = s * (x2 - 2.0 * xc + c2)
    m = jnp.max(SL, axis=0, keepdims=True)
    e = jnp.exp(SL - m)
    A = e / jnp.sum(e, axis=0, keepdims=True)
    Ech = jax.lax.dot_general(A, Xb, (((1,), (1,)), ((), ())),
                              preferred_element_type=jnp.float32)
    asum = jnp.sum(A, axis=1, keepdims=True)
    out_ref[0] = Ech - asum * C


def kernel(x, codewords, scale):
    b = x.shape[0]
    s2 = scale.reshape(_K, 1)
    out = pl.pallas_call(
        _enc_kernel,
        grid=(b,),
        in_specs=[
            pl.BlockSpec((1, _D, _H, _W), lambda bi: (bi, 0, 0, 0)),
            pl.BlockSpec((_K, _D), lambda bi: (0, 0)),
            pl.BlockSpec((_K, 1), lambda bi: (0, 0)),
        ],
        out_specs=pl.BlockSpec((1, _K, _D), lambda bi: (bi, 0, 0)),
        out_shape=jax.ShapeDtypeStruct((b, _K, _D), jnp.float32),
        compiler_params=pltpu.CompilerParams(
            dimension_semantics=("arbitrary",),
        ),
    )(x, codewords, s2)
    return out


# speedup vs baseline: 2.5224x; 2.5224x over previous
import jax
import jax.numpy as jnp
from jax.experimental import pallas as pl
from jax.experimental.pallas import tpu as pltpu

_D = 128
_K = 32
_NBLK = 4096


def _enc_kernel(xt_ref, cw_ref, scale_ref, out_ref):
    Xb = xt_ref[0]
    C = cw_ref[...]
    s = scale_ref[...]
    c2col = jnp.sum(C * C, axis=1, keepdims=True)
    c2 = c2col.reshape(1, _K)
    x2 = jnp.sum(Xb * Xb, axis=1, keepdims=True)
    xc = jax.lax.dot_general(Xb, C, (((1,), (1,)), ((), ())),
                             preferred_element_type=jnp.float32)
    SL = s * (x2 - 2.0 * xc + c2)
    m = jnp.max(SL, axis=1, keepdims=True)
    e = jnp.exp(SL - m)
    A = e / jnp.sum(e, axis=1, keepdims=True)
    Ech = jax.lax.dot_general(A, Xb, (((0,), (0,)), ((), ())),
                              preferred_element_type=jnp.float32)
    asum = jnp.sum(A, axis=0, keepdims=True)
    out_ref[0] = Ech - asum.reshape(_K, 1) * C


def kernel(x, codewords, scale):
    b, d, h, w = x.shape
    n_total = h * w
    xt = jnp.transpose(x, (0, 2, 3, 1)).reshape(b, n_total, d)
    s2 = scale.reshape(1, _K)
    out = pl.pallas_call(
        _enc_kernel,
        grid=(b,),
        in_specs=[
            pl.BlockSpec((1, _NBLK, _D), lambda bi: (bi, 0, 0)),
            pl.BlockSpec((_K, _D), lambda bi: (0, 0)),
            pl.BlockSpec((1, _K), lambda bi: (0, 0)),
        ],
        out_specs=pl.BlockSpec((1, _K, _D), lambda bi: (bi, 0, 0)),
        out_shape=jax.ShapeDtypeStruct((b, _K, _D), jnp.float32),
        compiler_params=pltpu.CompilerParams(
            dimension_semantics=("arbitrary",),
        ),
    )(xt, codewords, s2)
    return out


# speedup vs baseline: 4.2553x; 1.6870x over previous
import jax
import jax.numpy as jnp
from jax.experimental import pallas as pl
from jax.experimental.pallas import tpu as pltpu

_D = 128
_K = 32
_NBLK = 4096


def _enc_kernel(xt_ref, cw_ref, scale_ref, out_ref):
    Xb = xt_ref[0]
    C = cw_ref[...]
    s = scale_ref[...].reshape(_K, 1)
    c2 = jnp.sum(C * C, axis=1, keepdims=True)
    ones_row = jnp.ones((1, _D), dtype=jnp.float32)
    x2t = jax.lax.dot_general(ones_row, Xb * Xb, (((1,), (1,)), ((), ())),
                              preferred_element_type=jnp.float32)
    xct = jax.lax.dot_general(C, Xb, (((1,), (1,)), ((), ())),
                              preferred_element_type=jnp.float32)
    SL = s * (x2t - 2.0 * xct + c2)
    m = jnp.max(SL, axis=0, keepdims=True)
    e = jnp.exp(SL - m)
    A = e / jnp.sum(e, axis=0, keepdims=True)
    Ech = jax.lax.dot_general(A, Xb, (((1,), (0,)), ((), ())),
                              preferred_element_type=jnp.float32)
    asum = jnp.sum(A, axis=1, keepdims=True)
    out_ref[0] = Ech - asum * C


def kernel(x, codewords, scale):
    b, d, h, w = x.shape
    n_total = h * w
    xt = jnp.transpose(x, (0, 2, 3, 1)).reshape(b, n_total, d)
    s2 = scale.reshape(1, _K)
    out = pl.pallas_call(
        _enc_kernel,
        grid=(b,),
        in_specs=[
            pl.BlockSpec((1, _NBLK, _D), lambda bi: (bi, 0, 0)),
            pl.BlockSpec((_K, _D), lambda bi: (0, 0)),
            pl.BlockSpec((1, _K), lambda bi: (0, 0)),
        ],
        out_specs=pl.BlockSpec((1, _K, _D), lambda bi: (bi, 0, 0)),
        out_shape=jax.ShapeDtypeStruct((b, _K, _D), jnp.float32),
        compiler_params=pltpu.CompilerParams(
            dimension_semantics=("arbitrary",),
        ),
    )(xt, codewords, s2)
    return out
